# Initial kernel scaffold; baseline (speedup 1.0000x reference)
import jax
import jax.numpy as jnp
from jax.experimental import pallas as pl


def kernel(pred, target):
    raise NotImplementedError("write your pallas kernel here")


# speedup vs baseline: 6.5830x; 6.5830x over previous
import jax
import jax.numpy as jnp
from jax.experimental import pallas as pl


def _emd_greedy_kernel(pc, tx, ty, tz, out_ref):
    N, B, _ = pc.shape
    txv = tx[...]
    tyv = ty[...]
    tzv = tz[...]
    iota = jax.lax.broadcasted_iota(jnp.int32, (B, N), 1)
    inf = jnp.float32(jnp.inf)

    def step(i, carry):
        mask, acc = carry
        pci = pc[i]
        pxi = pci[:, 0:1]
        pyi = pci[:, 1:2]
        pzi = pci[:, 2:3]
        dx = pxi - txv
        dy = pyi - tyv
        dz = pzi - tzv
        d = dx * dx + dy * dy + dz * dz
        masked = d + mask
        m = jnp.min(masked, axis=1, keepdims=True)
        hit = masked <= m
        j = jnp.min(jnp.where(hit, iota, N), axis=1, keepdims=True)
        mask = jnp.where(iota == j, inf, mask)
        acc = acc + jnp.sqrt(m)
        return mask, acc

    mask0 = jnp.zeros((B, N), jnp.float32)
    acc0 = jnp.zeros((B, 1), jnp.float32)
    _, acc = jax.lax.fori_loop(0, N, step, (mask0, acc0))
    out_ref[...] = jnp.sum(acc, axis=(0, 1), keepdims=True) / jnp.float32(N * B)


def kernel(pred, target):
    B, N, _ = pred.shape
    pc = jnp.transpose(pred, (1, 0, 2))
    tx, ty, tz = target[..., 0], target[..., 1], target[..., 2]
    out = pl.pallas_call(
        _emd_greedy_kernel,
        out_shape=jax.ShapeDtypeStruct((1, 1), jnp.float32),
    )(pc, tx, ty, tz)
    return out[0, 0]


# speedup vs baseline: 23.6072x; 3.5861x over previous
import jax
import jax.numpy as jnp
from jax.experimental import pallas as pl
from jax.experimental.pallas import tpu as pltpu


def _emd_greedy_kernel(pxb, pyb, pzb, tx, ty, tz, out_ref, mask_ref, stale_ref):
    N = pxb.shape[0]
    G, B, L = tx.shape
    idx = (jax.lax.broadcasted_iota(jnp.int32, (G, B, L), 0) * L
           + jax.lax.broadcasted_iota(jnp.int32, (G, B, L), 2))
    inf = jnp.float32(jnp.inf)
    low = jnp.int32(2047)
    low_clear = jnp.int32(~2047)

    def dist_row(i):
        pxi = pxb[i][None, :, :]
        pyi = pyb[i][None, :, :]
        pzi = pzb[i][None, :, :]
        dx = pxi - tx[...]
        dy = pyi - ty[...]
        dz = pzi - tz[...]
        d = dx * dx + dy * dy + dz * dz
        bits = jax.lax.bitcast_convert_type(d, jnp.int32)
        return jax.lax.bitcast_convert_type((bits & low_clear) | idx, jnp.float32)

    def top2(sm):
        pairs = [(jnp.minimum(sm[2 * k], sm[2 * k + 1]),
                  jnp.maximum(sm[2 * k], sm[2 * k + 1])) for k in range(G // 2)]
        while len(pairs) > 1:
            nxt = []
            for k in range(0, len(pairs), 2):
                a1, a2 = pairs[k]
                b1, b2 = pairs[k + 1]
                nxt.append((jnp.minimum(a1, b1),
                            jnp.minimum(jnp.maximum(a1, b1), jnp.minimum(a2, b2))))
            pairs = nxt
        return pairs[0]

    def idx_of(v):
        return jax.lax.bitcast_convert_type(v, jnp.int32) & low

    d0 = dist_row(0)
    mask_ref[...] = jnp.zeros((G, B, L), jnp.float32)
    stale_ref[...] = d0
    p1_0, p2_0 = top2(d0)

    def step(i, carry):
        part, m_prev, acc = carry
        m = jnp.min(part, axis=1, keepdims=True)

        prev_cost = jnp.sqrt(jax.lax.bitcast_convert_type(
            jax.lax.bitcast_convert_type(m_prev, jnp.int32) & low_clear,
            jnp.float32))
        flag = jnp.where(i == 0, jnp.float32(0), jnp.float32(1))
        acc = acc + prev_cost * flag
        stale_i = stale_ref[...]
        mask_in = mask_ref[...]
        d_next = dist_row(jnp.minimum(i + 1, N - 1))
        stale_next = d_next + mask_in
        p1, p2 = top2(stale_next)

        mask_ref[...] = jnp.where(stale_i == m[None], inf, mask_in)
        stale_ref[...] = stale_next
        part_next = jnp.where(idx_of(p1) == idx_of(m), p2, p1)
        return part_next, m, acc

    acc0 = tx[...][0, :, 0:1] * jnp.float32(0)
    m_prev0 = acc0 + jnp.float32(4.0)
    part_last, m_last, acc = jax.lax.fori_loop(0, N, step, (p1_0, m_prev0, acc0))
    acc = acc + jnp.sqrt(jax.lax.bitcast_convert_type(
        jax.lax.bitcast_convert_type(m_last, jnp.int32) & low_clear, jnp.float32))
    out_ref[...] = jnp.sum(acc, axis=(0, 1), keepdims=True) / jnp.float32(N * B)


def kernel(pred, target):
    B, N, _ = pred.shape
    L = 128
    G = N // L
    pxb = jnp.broadcast_to(pred[..., 0].T[:, :, None], (N, B, L))
    pyb = jnp.broadcast_to(pred[..., 1].T[:, :, None], (N, B, L))
    pzb = jnp.broadcast_to(pred[..., 2].T[:, :, None], (N, B, L))
    tx = target[..., 0].reshape(B, G, L).transpose(1, 0, 2)
    ty = target[..., 1].reshape(B, G, L).transpose(1, 0, 2)
    tz = target[..., 2].reshape(B, G, L).transpose(1, 0, 2)
    out = pl.pallas_call(
        _emd_greedy_kernel,
        out_shape=jax.ShapeDtypeStruct((1, 1), jnp.float32),
        scratch_shapes=[
            pltpu.VMEM((G, B, L), jnp.float32),
            pltpu.VMEM((G, B, L), jnp.float32),
        ],
    )(pxb, pyb, pzb, tx, ty, tz)
    return out[0, 0]
